# baseline (speedup 1.0000x reference)
import functools

import jax
import jax.numpy as jnp
from jax import lax
from jax.experimental import pallas as pl
from jax.experimental.pallas import tpu as pltpu
from jax.experimental.pallas import tpu_sc as plsc

EMB_DIM = 64
CHUNK = 128
NBUF = 4
INFLIGHT = 3
NSPLIT = 2


@functools.lru_cache(maxsize=None)
def _build(num_flat, dim):
    mesh = plsc.VectorSubcoreMesh(core_axis_name="c", subcore_axis_name="s")
    nc, ns = mesh.num_cores, mesh.num_subcores
    nw = nc * ns
    assert num_flat % (nw * CHUNK) == 0
    nchunks = num_flat // (nw * CHUNK)
    assert nchunks % NBUF == 0 and nchunks >= NBUF

    @functools.partial(
        pl.kernel,
        out_type=jax.ShapeDtypeStruct((num_flat, dim), jnp.float32),
        mesh=mesh,
        scratch_types=[
            pltpu.VMEM((nchunks, CHUNK), jnp.int32),
            pltpu.VMEM((NBUF, CHUNK, dim), jnp.float32),
        ]
        + [pltpu.SemaphoreType.DMA] * (2 * NBUF),
        compiler_params=pltpu.CompilerParams(use_tc_tiling_on_sc=False),
    )
    def emb(idx_hbm, table_hbm, out_hbm, idx_v, rows_v, *sems):
        gsems, ssems = sems[:NBUF], sems[NBUF:]
        wid = lax.axis_index("s") * nc + lax.axis_index("c")
        base = wid * (nchunks * CHUNK)
        pltpu.sync_copy(idx_hbm.at[wid], idx_v)
        for b in range(INFLIGHT):
            pltpu.async_copy(table_hbm.at[idx_v.at[b]], rows_v.at[b], gsems[b])

        @pl.loop(0, nchunks, step=NBUF)
        def _(g):
            for b in range(NBUF):
                j = g + b
                pltpu.make_async_copy(
                    table_hbm.at[idx_v.at[b]], rows_v.at[b], gsems[b]
                ).wait()
                pltpu.async_copy(
                    rows_v.at[b], out_hbm.at[pl.ds(base + j * CHUNK, CHUNK)], ssems[b]
                )
                nj = j + INFLIGHT
                sb = (b + INFLIGHT) % NBUF

                @pl.when(nj < nchunks)
                def _():
                    @pl.when(nj >= NBUF)
                    def _():
                        pltpu.make_async_copy(
                            rows_v.at[sb],
                            out_hbm.at[pl.ds(base, CHUNK)],
                            ssems[sb],
                        ).wait()

                    pltpu.async_copy(
                        table_hbm.at[idx_v.at[nj]], rows_v.at[sb], gsems[sb]
                    )

        for b in range(NBUF):
            pltpu.make_async_copy(
                rows_v.at[b], out_hbm.at[pl.ds(base, CHUNK)], ssems[b]
            ).wait()

    return emb, nw, nchunks


def kernel(token_ids, weights):
    shape = token_ids.shape
    flat = token_ids.reshape(-1).astype(jnp.int32)
    half = flat.shape[0] // NSPLIT
    emb, nw, nchunks = _build(half, weights.shape[1])
    outs = [
        emb(flat[i * half:(i + 1) * half].reshape(nw, nchunks, CHUNK), weights)
        for i in range(NSPLIT)
    ]
    return jnp.concatenate(outs, axis=0).reshape(*shape, weights.shape[1])


# speedup vs baseline: 1.5062x; 1.5062x over previous
import functools

import jax
import jax.numpy as jnp
from jax import lax
from jax.experimental import pallas as pl
from jax.experimental.pallas import tpu as pltpu
from jax.experimental.pallas import tpu_sc as plsc

EMB_DIM = 64
CHUNK = 128
NBUF = 8
INFLIGHT = 6


@functools.lru_cache(maxsize=None)
def _build(num_flat, dim):
    mesh = plsc.VectorSubcoreMesh(core_axis_name="c", subcore_axis_name="s")
    nc, ns = mesh.num_cores, mesh.num_subcores
    nw = nc * ns
    assert num_flat % (nw * CHUNK) == 0
    nchunks = num_flat // (nw * CHUNK)
    assert nchunks % NBUF == 0 and nchunks >= NBUF

    @functools.partial(
        pl.kernel,
        out_type=jax.ShapeDtypeStruct((num_flat, dim), jnp.float32),
        mesh=mesh,
        scratch_types=[
            pltpu.VMEM((nchunks, CHUNK), jnp.int32),
            pltpu.VMEM((NBUF, CHUNK, dim), jnp.float32),
        ]
        + [pltpu.SemaphoreType.DMA] * (2 * NBUF),
        compiler_params=pltpu.CompilerParams(use_tc_tiling_on_sc=False),
    )
    def emb(idx_hbm, table_hbm, out_hbm, idx_v, rows_v, *sems):
        gsems, ssems = sems[:NBUF], sems[NBUF:]
        wid = lax.axis_index("s") * nc + lax.axis_index("c")
        base = wid * (nchunks * CHUNK)
        pltpu.sync_copy(idx_hbm.at[wid], idx_v)
        for b in range(INFLIGHT):
            pltpu.async_copy(table_hbm.at[idx_v.at[b]], rows_v.at[b], gsems[b])

        @pl.loop(0, nchunks, step=NBUF)
        def _(g):
            for b in range(NBUF):
                j = g + b
                pltpu.make_async_copy(
                    table_hbm.at[idx_v.at[b]], rows_v.at[b], gsems[b]
                ).wait()
                pltpu.async_copy(
                    rows_v.at[b], out_hbm.at[pl.ds(base + j * CHUNK, CHUNK)], ssems[b]
                )
                nj = j + INFLIGHT
                sb = (b + INFLIGHT) % NBUF

                @pl.when(nj < nchunks)
                def _():
                    @pl.when(nj >= NBUF)
                    def _():
                        pltpu.make_async_copy(
                            rows_v.at[sb],
                            out_hbm.at[pl.ds(base, CHUNK)],
                            ssems[sb],
                        ).wait()

                    pltpu.async_copy(
                        table_hbm.at[idx_v.at[nj]], rows_v.at[sb], gsems[sb]
                    )

        for b in range(NBUF):
            pltpu.make_async_copy(
                rows_v.at[b], out_hbm.at[pl.ds(base, CHUNK)], ssems[b]
            ).wait()

    return emb, nw, nchunks


def kernel(token_ids, weights):
    shape = token_ids.shape
    flat = token_ids.reshape(-1).astype(jnp.int32)
    emb, nw, nchunks = _build(flat.shape[0], weights.shape[1])
    idx3d = flat.reshape(nw, nchunks, CHUNK)
    out = emb(idx3d, weights)
    return out.reshape(*shape, weights.shape[1])
